# baseline (speedup 1.0000x reference)
import functools

import jax
import jax.numpy as jnp
from jax import lax
from jax.experimental import pallas as pl
from jax.experimental.pallas import tpu as pltpu
from jax.experimental.pallas import tpu_sc as plsc

NPTS = 8000
NT = 10000
R = 4096
SAMPLING_LENGTH = 0.05
MAX_SAMPLES = 128
BIG = 1.0e10

RB = 256
TB = 2048
RBS = 512


F_PAD = 40960

_SC_MESH = plsc.VectorSubcoreMesh(core_axis_name="c", subcore_axis_name="s")


def _sc_wid():
    return lax.axis_index("s") * 2 + lax.axis_index("c")


def _tri_gather(cage_flat, fids):
    ch = F_PAD // 32

    @functools.partial(
        pl.kernel,
        mesh=_SC_MESH,
        compiler_params=pltpu.CompilerParams(needs_layout_passes=False),
        out_type=jax.ShapeDtypeStruct((16, F_PAD), jnp.float32),
        scratch_types=[
            pltpu.VMEM((NPTS * 3,), jnp.float32),
            pltpu.VMEM((3, ch), jnp.int32),
            pltpu.VMEM((16, ch), jnp.float32),
        ],
    )
    def k(cage_hbm, fid_hbm, td_hbm, cage_v, fid_v, out_v):
        wid = _sc_wid()
        base = wid * ch
        pltpu.sync_copy(cage_hbm, cage_v)
        pltpu.sync_copy(fid_hbm.at[:, pl.ds(base, ch)], fid_v)

        def body(i, _):
            s = pl.ds(i * 16, 16)
            a0 = fid_v[0, s] * 3
            a1 = fid_v[1, s] * 3
            a2 = fid_v[2, s] * 3
            for c in range(3):
                x0 = plsc.load_gather(cage_v, [a0 + c])
                x1 = plsc.load_gather(cage_v, [a1 + c])
                x2 = plsc.load_gather(cage_v, [a2 + c])
                out_v[0 + c, s] = x0
                out_v[3 + c, s] = x1 - x0
                out_v[6 + c, s] = x2 - x0
            return 0

        lax.fori_loop(0, ch // 16, body, 0)
        pltpu.sync_copy(out_v, td_hbm.at[:, pl.ds(base, ch)])

    return k(cage_flat, fids)


def _tet_gather(fidx, t2t, tvid_flat, cage_flat):
    ch = R // 32
    nt = tvid_flat.shape[0] // 4
    nf = t2t.shape[0]

    @functools.partial(
        pl.kernel,
        mesh=_SC_MESH,
        compiler_params=pltpu.CompilerParams(needs_layout_passes=False),
        out_type=[
            jax.ShapeDtypeStruct((R,), jnp.int32),
            jax.ShapeDtypeStruct((R, 16), jnp.float32),
        ],
        scratch_types=[
            pltpu.VMEM((nf,), jnp.int32),
            pltpu.VMEM((4 * nt,), jnp.int32),
            pltpu.VMEM((NPTS * 3,), jnp.float32),
            pltpu.VMEM((ch,), jnp.int32),
            pltpu.VMEM((ch,), jnp.int32),
            pltpu.VMEM((ch, 16), jnp.float32),
        ],
    )
    def k(fidx_hbm, t2t_hbm, tvid_hbm, cage_hbm, tet_hbm, verts_hbm,
          t2t_v, tvid_v, cage_v, fidx_v, tet_v, verts_v):
        wid = _sc_wid()
        base = wid * ch
        pltpu.sync_copy(t2t_hbm, t2t_v)
        pltpu.sync_copy(tvid_hbm, tvid_v)
        pltpu.sync_copy(cage_hbm, cage_v)
        pltpu.sync_copy(fidx_hbm.at[pl.ds(base, ch)], fidx_v)

        def body(i, _):
            s = pl.ds(i * 16, 16)
            rows = i * 16 + lax.iota(jnp.int32, 16)
            f = fidx_v[s]
            tet = plsc.load_gather(t2t_v, [f])
            tet_v[s] = tet
            for kk in range(4):
                vid = plsc.load_gather(tvid_v, [tet + kk * nt])
                a = vid * 3
                for c in range(3):
                    comp = plsc.load_gather(cage_v, [a + c])
                    col = jnp.full((16,), 3 * kk + c, jnp.int32)
                    plsc.store_scatter(verts_v, [rows, col], comp)
            return 0

        lax.fori_loop(0, ch // 16, body, 0)
        pltpu.sync_copy(tet_v, tet_hbm.at[pl.ds(base, ch)])
        pltpu.sync_copy(verts_v, verts_hbm.at[pl.ds(base, ch), :])

    return k(fidx, t2t, tvid_flat, cage_flat)


def _intersect_kernel(od_ref, td_ref, dist_ref, fidx_ref, cur_ref, idx_ref,
                      *, nf):
    ft = pl.program_id(1)

    @pl.when(ft == 0)
    def _init():
        cur_ref[...] = jnp.full((RB, TB), BIG, jnp.float32)
        idx_ref[...] = jnp.zeros((RB, TB), jnp.int32)

    ox = od_ref[:, 0:1]
    oy = od_ref[:, 1:2]
    oz = od_ref[:, 2:3]
    dx = od_ref[:, 3:4]
    dy = od_ref[:, 4:5]
    dz = od_ref[:, 5:6]

    v0x = td_ref[0:1, :]
    v0y = td_ref[1:2, :]
    v0z = td_ref[2:3, :]
    e1x = td_ref[3:4, :]
    e1y = td_ref[4:5, :]
    e1z = td_ref[5:6, :]
    e2x = td_ref[6:7, :]
    e2y = td_ref[7:8, :]
    e2z = td_ref[8:9, :]

    px = dy * e2z - dz * e2y
    py = dz * e2x - dx * e2z
    pz = dx * e2y - dy * e2x
    det = e1x * px + e1y * py + e1z * pz
    ok = jnp.abs(det) > 1e-9
    inv = jnp.where(ok, 1.0 / det, 0.0)
    tx = ox - v0x
    ty = oy - v0y
    tz = oz - v0z
    uu = (tx * px + ty * py + tz * pz) * inv
    qx = ty * e1z - tz * e1y
    qy = tz * e1x - tx * e1z
    qz = tx * e1y - ty * e1x
    vv = (dx * qx + dy * qy + dz * qz) * inv
    tt = (e2x * qx + e2y * qy + e2z * qz) * inv
    hit = (uu >= 0.0) & (vv >= 0.0) & (uu + vv <= 1.0) & (tt > 1e-6)
    tval = jnp.where(hit, tt, BIG)

    idx_row = ft * TB + jax.lax.broadcasted_iota(jnp.int32, (1, TB), 1)
    better = tval < cur_ref[...]
    cur_ref[...] = jnp.where(better, tval, cur_ref[...])
    idx_ref[...] = jnp.where(better, idx_row, idx_ref[...])

    @pl.when(ft == nf - 1)
    def _done():
        cv = cur_ref[...]
        m = jnp.min(cv, axis=1)
        sel = jnp.where(cv == m[:, None], idx_ref[...], jnp.int32(2**31 - 1))
        dist_ref[...] = m
        fidx_ref[...] = jnp.min(sel, axis=1)


def _intersect(od, td):
    nf = td.shape[1] // TB
    grid = (R // RB, nf)
    return pl.pallas_call(
        functools.partial(_intersect_kernel, nf=nf),
        grid=grid,
        in_specs=[
            pl.BlockSpec((RB, 8), lambda r, f: (r, 0)),
            pl.BlockSpec((16, TB), lambda r, f: (0, f)),
        ],
        out_specs=[
            pl.BlockSpec((RB,), lambda r, f: (r,)),
            pl.BlockSpec((RB,), lambda r, f: (r,)),
        ],
        out_shape=[
            jax.ShapeDtypeStruct((R,), jnp.float32),
            jax.ShapeDtypeStruct((R,), jnp.int32),
        ],
        scratch_shapes=[
            pltpu.VMEM((RB, TB), jnp.float32),
            pltpu.VMEM((RB, TB), jnp.int32),
        ],
    )(od, td)


def _sampler_kernel(od_ref, dist_ref, tet_ref, verts_ref,
                    ray_idx_ref, tet_idx_ref, t0_ref, t1_ref,
                    b0_ref, b1_ref, b2_ref, b3_ref,
                    px_ref, py_ref, pz_ref):
    rb = pl.program_id(0)
    dist = dist_ref[...]
    mask = dist < 5.0
    safe = jnp.where(mask, dist, 0.0)

    base = rb * RBS
    ray_iota = base + jax.lax.broadcasted_iota(jnp.int32, (RBS,), 0)
    ray_idx_ref[...] = jnp.where(mask, ray_iota, -1)
    tet_idx_ref[...] = jnp.where(mask, tet_ref[...], -1)
    t0_ref[...] = safe
    t1_ref[...] = jnp.where(mask, safe + SAMPLING_LENGTH, 0.0)

    ox = od_ref[:, 0:1]
    oy = od_ref[:, 1:2]
    oz = od_ref[:, 2:3]
    dx = od_ref[:, 3:4]
    dy = od_ref[:, 4:5]
    dz = od_ref[:, 5:6]
    sd = safe[:, None]
    hx = ox + sd * dx
    hy = oy + sd * dy
    hz = oz + sd * dz

    step = SAMPLING_LENGTH / MAX_SAMPLES
    offs = step * jax.lax.broadcasted_iota(
        jnp.int32, (1, MAX_SAMPLES), 1).astype(jnp.float32)
    px = hx + offs * dx
    py = hy + offs * dy
    pz = hz + offs * dz

    w0x = verts_ref[:, 0:1]
    w0y = verts_ref[:, 1:2]
    w0z = verts_ref[:, 2:3]
    m00 = verts_ref[:, 3:4] - w0x
    m10 = verts_ref[:, 4:5] - w0y
    m20 = verts_ref[:, 5:6] - w0z
    m01 = verts_ref[:, 6:7] - w0x
    m11 = verts_ref[:, 7:8] - w0y
    m21 = verts_ref[:, 8:9] - w0z
    m02 = verts_ref[:, 9:10] - w0x
    m12 = verts_ref[:, 10:11] - w0y
    m22 = verts_ref[:, 11:12] - w0z
    m00 = m00 + 1e-5
    m11 = m11 + 1e-5
    m22 = m22 + 1e-5

    rx = px - w0x
    ry = py - w0y
    rz = pz - w0z

    a0 = jnp.abs(m00)
    a1 = jnp.abs(m10)
    a2 = jnp.abs(m20)
    s2 = a2 > jnp.maximum(a0, a1)
    t1 = (a1 > a0) & jnp.logical_not(s2)
    n00 = jnp.where(s2, m20, jnp.where(t1, m10, m00))
    n01 = jnp.where(s2, m21, jnp.where(t1, m11, m01))
    n02 = jnp.where(s2, m22, jnp.where(t1, m12, m02))
    n10 = jnp.where(t1, m00, m10)
    n11 = jnp.where(t1, m01, m11)
    n12 = jnp.where(t1, m02, m12)
    n20 = jnp.where(s2, m00, m20)
    n21 = jnp.where(s2, m01, m21)
    n22 = jnp.where(s2, m02, m22)
    q0x = jnp.where(s2, rz, jnp.where(t1, ry, rx))
    q0y = jnp.where(t1, rx, ry)
    q0z = jnp.where(s2, rx, rz)

    l10 = n10 / n00
    l20 = n20 / n00
    n11 = n11 - l10 * n01
    n12 = n12 - l10 * n02
    n21 = n21 - l20 * n01
    n22 = n22 - l20 * n02
    q1y = q0y - l10 * q0x
    q1z = q0z - l20 * q0x

    s3 = jnp.abs(n21) > jnp.abs(n11)
    p11 = jnp.where(s3, n21, n11)
    p12 = jnp.where(s3, n22, n12)
    p21 = jnp.where(s3, n11, n21)
    p22 = jnp.where(s3, n12, n22)
    q2y = jnp.where(s3, q1z, q1y)
    q2z = jnp.where(s3, q1y, q1z)

    l21 = p21 / p11
    p22 = p22 - l21 * p12
    q3z = q2z - l21 * q2y

    b3 = q3z / p22
    b2 = (q2y - p12 * b3) / p11
    b1 = (q0x - n01 * b2 - n02 * b3) / n00
    b0 = 1.0 - (b1 + b2 + b3)

    fm = mask.astype(jnp.float32)[:, None]
    b0_ref[...] = b0 * fm
    b1_ref[...] = b1 * fm
    b2_ref[...] = b2 * fm
    b3_ref[...] = b3 * fm
    px_ref[...] = px * fm
    py_ref[...] = py * fm
    pz_ref[...] = pz * fm


def _sampler(od, dist, tet, verts):
    grid = (R // RBS,)
    return pl.pallas_call(
        _sampler_kernel,
        grid=grid,
        in_specs=[
            pl.BlockSpec((RBS, 8), lambda r: (r, 0)),
            pl.BlockSpec((RBS,), lambda r: (r,)),
            pl.BlockSpec((RBS,), lambda r: (r,)),
            pl.BlockSpec((RBS, 16), lambda r: (r, 0)),
        ],
        out_specs=(
            [pl.BlockSpec((RBS,), lambda r: (r,))] * 4
            + [pl.BlockSpec((RBS, MAX_SAMPLES), lambda r: (r, 0))] * 7
        ),
        out_shape=(
            [jax.ShapeDtypeStruct((R,), jnp.int32)] * 2
            + [jax.ShapeDtypeStruct((R,), jnp.float32)] * 2
            + [jax.ShapeDtypeStruct((R, MAX_SAMPLES), jnp.float32)] * 7
        ),
    )(od, dist, tet, verts)


def kernel(cage_vertices, ro, rd, ABCD, triangle_to_tetra, topology):
    cage_flat = cage_vertices[0].reshape(-1)
    o = ro[0]
    d = rd[0]
    od = jnp.concatenate([o, d, jnp.zeros((R, 2), jnp.float32)], axis=1)

    faces = ABCD.reshape(-1, 3)
    F = faces.shape[0]
    fids = jnp.pad(faces.T, ((0, 0), (0, F_PAD - F)))
    td = _tri_gather(cage_flat, fids)

    dist, fidx = _intersect(od, td)

    tvid_flat = jnp.concatenate(
        [ABCD[:, 0, 0], ABCD[:, 0, 1], ABCD[:, 0, 2], ABCD[:, 1, 2]])
    closest_tetras, verts = _tet_gather(
        fidx, triangle_to_tetra, tvid_flat, cage_flat)

    (ray_idx, tet_idx, t0, t1,
     b0, b1, b2, b3, px, py, pz) = _sampler(od, dist, closest_tetras, verts)
    bary = jnp.stack([b0, b1, b2, b3], axis=-1)
    pos = jnp.stack([px, py, pz], axis=-1)
    return ray_idx, tet_idx, bary, t0, t1, pos


# speedup vs baseline: 2.0216x; 2.0216x over previous
import functools

import jax
import jax.numpy as jnp
from jax import lax
from jax.experimental import pallas as pl
from jax.experimental.pallas import tpu as pltpu
from jax.experimental.pallas import tpu_sc as plsc

NPTS = 8000
NT = 10000
R = 4096
SAMPLING_LENGTH = 0.05
MAX_SAMPLES = 128
BIG = 1.0e10

RB = 256
TB = 1024
RBS = 512


F_PAD = 40960

_SC_MESH = plsc.VectorSubcoreMesh(core_axis_name="c", subcore_axis_name="s")


def _sc_wid():
    return lax.axis_index("s") * 2 + lax.axis_index("c")


def _tri_gather(cage_flat, fids):
    ch = F_PAD // 32

    @functools.partial(
        pl.kernel,
        mesh=_SC_MESH,
        compiler_params=pltpu.CompilerParams(needs_layout_passes=False),
        out_type=jax.ShapeDtypeStruct((16, F_PAD), jnp.float32),
        scratch_types=[
            pltpu.VMEM((NPTS * 3,), jnp.float32),
            pltpu.VMEM((3, ch), jnp.int32),
            pltpu.VMEM((16, ch), jnp.float32),
        ],
    )
    def k(cage_hbm, fid_hbm, td_hbm, cage_v, fid_v, out_v):
        wid = _sc_wid()
        base = wid * ch
        pltpu.sync_copy(cage_hbm, cage_v)
        pltpu.sync_copy(fid_hbm.at[:, pl.ds(base, ch)], fid_v)

        def body(i, _):
            s = pl.ds(i * 16, 16)
            a0 = fid_v[0, s] * 3
            a1 = fid_v[1, s] * 3
            a2 = fid_v[2, s] * 3
            for c in range(3):
                x0 = plsc.load_gather(cage_v, [a0 + c])
                x1 = plsc.load_gather(cage_v, [a1 + c])
                x2 = plsc.load_gather(cage_v, [a2 + c])
                out_v[0 + c, s] = x0
                out_v[3 + c, s] = x1 - x0
                out_v[6 + c, s] = x2 - x0
            return 0

        lax.fori_loop(0, ch // 16, body, 0)
        pltpu.sync_copy(out_v, td_hbm.at[:, pl.ds(base, ch)])

    return k(cage_flat, fids)


def _tet_gather(fidx, t2t, tvid_flat, cage_flat):
    ch = R // 32
    nt = tvid_flat.shape[0] // 4
    nf = t2t.shape[0]

    @functools.partial(
        pl.kernel,
        mesh=_SC_MESH,
        compiler_params=pltpu.CompilerParams(needs_layout_passes=False),
        out_type=[
            jax.ShapeDtypeStruct((R,), jnp.int32),
            jax.ShapeDtypeStruct((R, 16), jnp.float32),
        ],
        scratch_types=[
            pltpu.VMEM((nf,), jnp.int32),
            pltpu.VMEM((4 * nt,), jnp.int32),
            pltpu.VMEM((NPTS * 3,), jnp.float32),
            pltpu.VMEM((ch,), jnp.int32),
            pltpu.VMEM((ch,), jnp.int32),
            pltpu.VMEM((ch, 16), jnp.float32),
        ],
    )
    def k(fidx_hbm, t2t_hbm, tvid_hbm, cage_hbm, tet_hbm, verts_hbm,
          t2t_v, tvid_v, cage_v, fidx_v, tet_v, verts_v):
        wid = _sc_wid()
        base = wid * ch
        pltpu.sync_copy(t2t_hbm, t2t_v)
        pltpu.sync_copy(tvid_hbm, tvid_v)
        pltpu.sync_copy(cage_hbm, cage_v)
        pltpu.sync_copy(fidx_hbm.at[pl.ds(base, ch)], fidx_v)

        def body(i, _):
            s = pl.ds(i * 16, 16)
            rows = i * 16 + lax.iota(jnp.int32, 16)
            f = fidx_v[s]
            tet = plsc.load_gather(t2t_v, [f])
            tet_v[s] = tet
            for kk in range(4):
                vid = plsc.load_gather(tvid_v, [tet + kk * nt])
                a = vid * 3
                for c in range(3):
                    comp = plsc.load_gather(cage_v, [a + c])
                    col = jnp.full((16,), 3 * kk + c, jnp.int32)
                    plsc.store_scatter(verts_v, [rows, col], comp)
            return 0

        lax.fori_loop(0, ch // 16, body, 0)
        pltpu.sync_copy(tet_v, tet_hbm.at[pl.ds(base, ch)])
        pltpu.sync_copy(verts_v, verts_hbm.at[pl.ds(base, ch), :])

    return k(fidx, t2t, tvid_flat, cage_flat)


def _intersect_kernel(od_ref, td_ref, dist_ref, fidx_ref, cur_ref, idx_ref,
                      *, nf):
    ft = pl.program_id(1)

    @pl.when(ft == 0)
    def _init():
        cur_ref[...] = jnp.full((RB, TB), BIG, jnp.float32)
        idx_ref[...] = jnp.zeros((RB, TB), jnp.int32)

    ox = od_ref[:, 0:1]
    oy = od_ref[:, 1:2]
    oz = od_ref[:, 2:3]
    dx = od_ref[:, 3:4]
    dy = od_ref[:, 4:5]
    dz = od_ref[:, 5:6]

    v0x = td_ref[0:1, :]
    v0y = td_ref[1:2, :]
    v0z = td_ref[2:3, :]
    e1x = td_ref[3:4, :]
    e1y = td_ref[4:5, :]
    e1z = td_ref[5:6, :]
    e2x = td_ref[6:7, :]
    e2y = td_ref[7:8, :]
    e2z = td_ref[8:9, :]

    px = dy * e2z - dz * e2y
    py = dz * e2x - dx * e2z
    pz = dx * e2y - dy * e2x
    det = e1x * px + e1y * py + e1z * pz
    ok = jnp.abs(det) > 1e-9
    inv = jnp.where(ok, 1.0 / det, 0.0)
    tx = ox - v0x
    ty = oy - v0y
    tz = oz - v0z
    uu = (tx * px + ty * py + tz * pz) * inv
    qx = ty * e1z - tz * e1y
    qy = tz * e1x - tx * e1z
    qz = tx * e1y - ty * e1x
    vv = (dx * qx + dy * qy + dz * qz) * inv
    tt = (e2x * qx + e2y * qy + e2z * qz) * inv
    hit = (uu >= 0.0) & (vv >= 0.0) & (uu + vv <= 1.0) & (tt > 1e-6)
    tval = jnp.where(hit, tt, BIG)

    idx_row = ft * TB + jax.lax.broadcasted_iota(jnp.int32, (1, TB), 1)
    better = tval < cur_ref[...]
    cur_ref[...] = jnp.where(better, tval, cur_ref[...])
    idx_ref[...] = jnp.where(better, idx_row, idx_ref[...])

    @pl.when(ft == nf - 1)
    def _done():
        cv = cur_ref[...]
        m = jnp.min(cv, axis=1)
        sel = jnp.where(cv == m[:, None], idx_ref[...], jnp.int32(2**31 - 1))
        dist_ref[...] = m
        fidx_ref[...] = jnp.min(sel, axis=1)


def _intersect(od, td):
    nf = td.shape[1] // TB
    grid = (R // RB, nf)
    return pl.pallas_call(
        functools.partial(_intersect_kernel, nf=nf),
        grid=grid,
        in_specs=[
            pl.BlockSpec((RB, 8), lambda r, f: (r, 0)),
            pl.BlockSpec((16, TB), lambda r, f: (0, f)),
        ],
        out_specs=[
            pl.BlockSpec((RB,), lambda r, f: (r,)),
            pl.BlockSpec((RB,), lambda r, f: (r,)),
        ],
        out_shape=[
            jax.ShapeDtypeStruct((R,), jnp.float32),
            jax.ShapeDtypeStruct((R,), jnp.int32),
        ],
        scratch_shapes=[
            pltpu.VMEM((RB, TB), jnp.float32),
            pltpu.VMEM((RB, TB), jnp.int32),
        ],
    )(od, td)


def _sampler_kernel(od_ref, dist_ref, tet_ref, verts_ref,
                    ray_idx_ref, tet_idx_ref, t0_ref, t1_ref,
                    b0_ref, b1_ref, b2_ref, b3_ref,
                    px_ref, py_ref, pz_ref):
    rb = pl.program_id(0)
    dist = dist_ref[...]
    mask = dist < 5.0
    safe = jnp.where(mask, dist, 0.0)

    base = rb * RBS
    ray_iota = base + jax.lax.broadcasted_iota(jnp.int32, (RBS,), 0)
    ray_idx_ref[...] = jnp.where(mask, ray_iota, -1)
    tet_idx_ref[...] = jnp.where(mask, tet_ref[...], -1)
    t0_ref[...] = safe
    t1_ref[...] = jnp.where(mask, safe + SAMPLING_LENGTH, 0.0)

    ox = od_ref[:, 0:1]
    oy = od_ref[:, 1:2]
    oz = od_ref[:, 2:3]
    dx = od_ref[:, 3:4]
    dy = od_ref[:, 4:5]
    dz = od_ref[:, 5:6]
    sd = safe[:, None]
    hx = ox + sd * dx
    hy = oy + sd * dy
    hz = oz + sd * dz

    step = SAMPLING_LENGTH / MAX_SAMPLES
    offs = step * jax.lax.broadcasted_iota(
        jnp.int32, (1, MAX_SAMPLES), 1).astype(jnp.float32)
    px = hx + offs * dx
    py = hy + offs * dy
    pz = hz + offs * dz

    w0x = verts_ref[:, 0:1]
    w0y = verts_ref[:, 1:2]
    w0z = verts_ref[:, 2:3]
    m00 = verts_ref[:, 3:4] - w0x
    m10 = verts_ref[:, 4:5] - w0y
    m20 = verts_ref[:, 5:6] - w0z
    m01 = verts_ref[:, 6:7] - w0x
    m11 = verts_ref[:, 7:8] - w0y
    m21 = verts_ref[:, 8:9] - w0z
    m02 = verts_ref[:, 9:10] - w0x
    m12 = verts_ref[:, 10:11] - w0y
    m22 = verts_ref[:, 11:12] - w0z
    m00 = m00 + 1e-5
    m11 = m11 + 1e-5
    m22 = m22 + 1e-5

    rx = px - w0x
    ry = py - w0y
    rz = pz - w0z

    a0 = jnp.abs(m00)
    a1 = jnp.abs(m10)
    a2 = jnp.abs(m20)
    s2 = a2 > jnp.maximum(a0, a1)
    t1 = (a1 > a0) & jnp.logical_not(s2)
    n00 = jnp.where(s2, m20, jnp.where(t1, m10, m00))
    n01 = jnp.where(s2, m21, jnp.where(t1, m11, m01))
    n02 = jnp.where(s2, m22, jnp.where(t1, m12, m02))
    n10 = jnp.where(t1, m00, m10)
    n11 = jnp.where(t1, m01, m11)
    n12 = jnp.where(t1, m02, m12)
    n20 = jnp.where(s2, m00, m20)
    n21 = jnp.where(s2, m01, m21)
    n22 = jnp.where(s2, m02, m22)
    q0x = jnp.where(s2, rz, jnp.where(t1, ry, rx))
    q0y = jnp.where(t1, rx, ry)
    q0z = jnp.where(s2, rx, rz)

    l10 = n10 / n00
    l20 = n20 / n00
    n11 = n11 - l10 * n01
    n12 = n12 - l10 * n02
    n21 = n21 - l20 * n01
    n22 = n22 - l20 * n02
    q1y = q0y - l10 * q0x
    q1z = q0z - l20 * q0x

    s3 = jnp.abs(n21) > jnp.abs(n11)
    p11 = jnp.where(s3, n21, n11)
    p12 = jnp.where(s3, n22, n12)
    p21 = jnp.where(s3, n11, n21)
    p22 = jnp.where(s3, n12, n22)
    q2y = jnp.where(s3, q1z, q1y)
    q2z = jnp.where(s3, q1y, q1z)

    l21 = p21 / p11
    p22 = p22 - l21 * p12
    q3z = q2z - l21 * q2y

    b3 = q3z / p22
    b2 = (q2y - p12 * b3) / p11
    b1 = (q0x - n01 * b2 - n02 * b3) / n00
    b0 = 1.0 - (b1 + b2 + b3)

    fm = mask.astype(jnp.float32)[:, None]
    b0_ref[...] = b0 * fm
    b1_ref[...] = b1 * fm
    b2_ref[...] = b2 * fm
    b3_ref[...] = b3 * fm
    px_ref[...] = px * fm
    py_ref[...] = py * fm
    pz_ref[...] = pz * fm


def _sampler(od, dist, tet, verts):
    grid = (R // RBS,)
    return pl.pallas_call(
        _sampler_kernel,
        grid=grid,
        in_specs=[
            pl.BlockSpec((RBS, 8), lambda r: (r, 0)),
            pl.BlockSpec((RBS,), lambda r: (r,)),
            pl.BlockSpec((RBS,), lambda r: (r,)),
            pl.BlockSpec((RBS, 16), lambda r: (r, 0)),
        ],
        out_specs=(
            [pl.BlockSpec((RBS,), lambda r: (r,))] * 4
            + [pl.BlockSpec((RBS, MAX_SAMPLES), lambda r: (r, 0))] * 7
        ),
        out_shape=(
            [jax.ShapeDtypeStruct((R,), jnp.int32)] * 2
            + [jax.ShapeDtypeStruct((R,), jnp.float32)] * 2
            + [jax.ShapeDtypeStruct((R, MAX_SAMPLES), jnp.float32)] * 7
        ),
    )(od, dist, tet, verts)


def kernel(cage_vertices, ro, rd, ABCD, triangle_to_tetra, topology):
    cage_flat = cage_vertices[0].reshape(-1)
    o = ro[0]
    d = rd[0]
    od = jnp.concatenate([o, d, jnp.zeros((R, 2), jnp.float32)], axis=1)

    faces = ABCD.reshape(-1, 3)
    F = faces.shape[0]
    fids = jnp.pad(faces.T, ((0, 0), (0, F_PAD - F)))
    td = _tri_gather(cage_flat, fids)

    dist, fidx = _intersect(od, td)

    tvid_flat = jnp.concatenate(
        [ABCD[:, 0, 0], ABCD[:, 0, 1], ABCD[:, 0, 2], ABCD[:, 1, 2]])
    closest_tetras, verts = _tet_gather(
        fidx, triangle_to_tetra, tvid_flat, cage_flat)

    (ray_idx, tet_idx, t0, t1,
     b0, b1, b2, b3, px, py, pz) = _sampler(od, dist, closest_tetras, verts)
    bary = jnp.stack([b0, b1, b2, b3], axis=-1)
    pos = jnp.stack([px, py, pz], axis=-1)
    return ray_idx, tet_idx, bary, t0, t1, pos
